# baseline (speedup 1.0000x reference)
import functools
import jax
import jax.numpy as jnp
from jax import lax
from jax.experimental import pallas as pl
from jax.experimental.pallas import tpu as pltpu, tpu_sc as plsc

_N = 10000
_E = 160000
_EMB = 128
_ATTN = 32
_NUM_RELS = 8
_NUM_LAYERS = 3

_ROWS_BLK = 1000
_NB = _N // _ROWS_BLK

_NC = 2
_NS = 16
_NW = _NC * _NS
_CH = 32
_NCH = 158
_EPT = _CH * _NCH
_E_PAD = _EPT * _NW
_RPS = 624
_N_TAIL = _N - _RPS * _NS


def _mm_kernel(h_ref, w_ref, o_ref):
    o_ref[0] = jnp.dot(h_ref[...], w_ref[0], preferred_element_type=jnp.float32)


def _stacked_matmul(h, w_all):
    c = w_all.shape[0]
    return pl.pallas_call(
        _mm_kernel,
        grid=(c, _NB),
        in_specs=[
            pl.BlockSpec((_ROWS_BLK, _EMB), lambda i, j: (j, 0)),
            pl.BlockSpec((1, _EMB, _EMB), lambda i, j: (i, 0, 0)),
        ],
        out_specs=pl.BlockSpec((1, _ROWS_BLK, _EMB), lambda i, j: (i, j, 0)),
        out_shape=jax.ShapeDtypeStruct((c, _N, _EMB), jnp.float32),
    )(h, w_all)


def _comb_kernel(a1_ref, relb_ref, o_ref):
    o_ref[0] = a1_ref[...] + relb_ref[0, 0]


def _build_comb(hA1, relb):
    return pl.pallas_call(
        _comb_kernel,
        grid=(_NUM_RELS, _NB),
        in_specs=[
            pl.BlockSpec((_ROWS_BLK, _EMB), lambda i, j: (j, 0)),
            pl.BlockSpec((1, 1, _EMB), lambda i, j: (i, 0, 0)),
        ],
        out_specs=pl.BlockSpec((1, _ROWS_BLK, _EMB), lambda i, j: (i, j, 0)),
        out_shape=jax.ShapeDtypeStruct((_NUM_RELS, _N, _EMB), jnp.float32),
    )(hA1, relb[:, None, :])


def _post_kernel(hs_ref, agg_ref, bias_ref, o_ref):
    o_ref[...] = jnp.maximum(
        hs_ref[...] + agg_ref[0, 0] + agg_ref[0, 1] + bias_ref[...], 0.0)


def _post_layer(hs, agg2, bias_l):
    return pl.pallas_call(
        _post_kernel,
        grid=(_NB,),
        in_specs=[
            pl.BlockSpec((_ROWS_BLK, _EMB), lambda j: (j, 0)),
            pl.BlockSpec((1, 2, _ROWS_BLK, _EMB), lambda j: (0, 0, j, 0)),
            pl.BlockSpec((1, _EMB), lambda j: (0, 0)),
        ],
        out_specs=pl.BlockSpec((_ROWS_BLK, _EMB), lambda j: (j, 0)),
        out_shape=jax.ShapeDtypeStruct((_N, _EMB), jnp.float32),
    )(hs, agg2[None], bias_l[None])


def _stats_kernel(h_ref, o_ref):
    @pl.when(pl.program_id(0) == 0)
    def _():
        o_ref[...] = jnp.zeros_like(o_ref)
    blk = h_ref[...]
    o_ref[0, 0] += jnp.sum(blk, axis=0)
    o_ref[0, 1] += jnp.sum(blk * blk, axis=0)


def _bn_stats(h):
    return pl.pallas_call(
        _stats_kernel,
        grid=(_NB,),
        in_specs=[pl.BlockSpec((_ROWS_BLK, _EMB), lambda j: (j, 0))],
        out_specs=pl.BlockSpec((1, 2, _EMB), lambda j: (0, 0, 0)),
        out_shape=jax.ShapeDtypeStruct((1, 2, _EMB), jnp.float32),
    )(h)


def _bn_apply_kernel(h_ref, hin_ref, scale_ref, shift_ref, o_ref):
    o_ref[...] = hin_ref[...] + h_ref[...] * scale_ref[0] + shift_ref[0]


def _bn_apply(h, h_in, scale, shift):
    return pl.pallas_call(
        _bn_apply_kernel,
        grid=(_NB,),
        in_specs=[
            pl.BlockSpec((_ROWS_BLK, _EMB), lambda j: (j, 0)),
            pl.BlockSpec((_ROWS_BLK, _EMB), lambda j: (j, 0)),
            pl.BlockSpec((1, _EMB), lambda j: (0, 0)),
            pl.BlockSpec((1, _EMB), lambda j: (0, 0)),
        ],
        out_specs=pl.BlockSpec((_ROWS_BLK, _EMB), lambda j: (j, 0)),
        out_shape=jax.ShapeDtypeStruct((_N, _EMB), jnp.float32),
    )(h, h_in, scale[None], shift[None])


_G16 = _EMB // 16


def _edge_sc_kernel(hW_hbm, comb_hbm, hA2_hbm, gidx_hbm, dst_hbm,
                    norm_hbm, bw_hbm, bb_hbm, out_hbm,
                    gidx_v, dst_v,
                    msg0, msg1, a10, a11, a20, a21, out0, out1,
                    dst0, dst1, nrm0, nrm1, tbuf, vbuf,
                    bw_v, bb_v, zbuf, agg_sh,
                    s0a, s0b, s0c, s0d, s1a, s1b, s1c, s1d, ssc0, ssc1):
    cid = lax.axis_index("c")
    sid = lax.axis_index("s")
    wid = sid * _NC + cid
    ebase = wid * _EPT

    pltpu.sync_copy(gidx_hbm.at[pl.ds(ebase, _EPT)], gidx_v)
    pltpu.sync_copy(dst_hbm.at[pl.ds(ebase, _EPT)], dst_v)
    pltpu.sync_copy(bw_hbm, bw_v)
    pltpu.sync_copy(bb_hbm, bb_v)

    def _zb_body(k, _):
        for j in range(_G16):
            zbuf[k, pl.ds(j * 16, 16)] = jnp.zeros((16,), jnp.float32)
        return 0
    lax.fori_loop(0, 8, _zb_body, 0)

    def _zc_body(k, _):
        pltpu.sync_copy(zbuf, agg_sh.at[pl.ds(sid * _RPS + k * 8, 8)])
        return 0
    lax.fori_loop(0, _RPS // 8, _zc_body, 0)

    @pl.when(sid == _NS - 1)
    def _zero_tail():
        pltpu.sync_copy(zbuf, agg_sh.at[pl.ds(_RPS * _NS, 8)])
        pltpu.sync_copy(zbuf, agg_sh.at[pl.ds(_RPS * _NS + 8, 8)])
    plsc.subcore_barrier()

    bwv = [bw_v[pl.ds(j * 16, 16)] for j in range(_G16)]
    bbv = bb_v[...]
    eidx = lax.iota(jnp.int32, 16)
    msgb = (msg0, msg1)
    a1b = (a10, a11)
    a2b = (a20, a21)
    outb = (out0, out1)
    dstb = (dst0, dst1)
    nrmb = (nrm0, nrm1)
    sscb = (ssc0, ssc1)
    sems = ((s0a, s0b, s0c, s0d), (s1a, s1b, s1c, s1d))

    def _fire(b, ci):
        cb = ci * _CH
        idx_m = gidx_v.at[pl.ds(cb, _CH)]
        idx_d = dst_v.at[pl.ds(cb, _CH)]
        pltpu.async_copy(hW_hbm.at[idx_m], msgb[b], sems[b][0])
        pltpu.async_copy(comb_hbm.at[idx_m], a1b[b], sems[b][1])
        pltpu.async_copy(hA2_hbm.at[idx_d], a2b[b], sems[b][2])
        pltpu.async_copy(norm_hbm.at[pl.ds(ebase + cb, _CH)], nrmb[b],
                         sems[b][3])

    def _wait(b, ci):
        cb = ci * _CH
        idx_m = gidx_v.at[pl.ds(cb, _CH)]
        idx_d = dst_v.at[pl.ds(cb, _CH)]
        pltpu.make_async_copy(hW_hbm.at[idx_m], msgb[b], sems[b][0]).wait()
        pltpu.make_async_copy(comb_hbm.at[idx_m], a1b[b], sems[b][1]).wait()
        pltpu.make_async_copy(hA2_hbm.at[idx_d], a2b[b], sems[b][2]).wait()
        pltpu.make_async_copy(norm_hbm.at[pl.ds(ebase + cb, _CH)], nrmb[b],
                              sems[b][3]).wait()

    def _process(b, ci):
        msgc, a1c, a2c = msgb[b], a1b[b], a2b[b]
        outc, dstc, ssc, nrmc = outb[b], dstb[b], sscb[b], nrmb[b]
        cb = ci * _CH
        for g in range(_CH // 16):
            def _edge_dot(ee, _, g=g):
                e = g * 16 + ee
                vacc = jnp.zeros((16,), jnp.float32)
                for j in range(_G16):
                    sl = pl.ds(j * 16, 16)
                    pre = jnp.maximum(a1c[e, sl] + a2c[e, sl], 0.0)
                    vacc = vacc + pre * bwv[j]
                vbuf[ee, :] = vacc
                return 0
            lax.fori_loop(0, 16, _edge_dot, 0, unroll=4)
            tv = jnp.zeros((16,), jnp.float32)
            for j in range(16):
                tv = tv + plsc.load_gather(
                    vbuf, [eidx, jnp.full((16,), j, jnp.int32)])
            av = 1.0 / (1.0 + jnp.exp(-(tv + bbv)))
            tbuf[pl.ds(g * 16, 16)] = av * nrmc[pl.ds(g * 16, 16)]

        @pl.when(ci >= 2)
        def _():
            pltpu.make_async_copy(outc, agg_sh.at[dstc], ssc).wait()

        def _edge_scale(e, _):
            sc = plsc.load_gather(tbuf, [jnp.full((16,), e, jnp.int32)])
            for j in range(_G16):
                sl = pl.ds(j * 16, 16)
                outc[e, sl] = msgc[e, sl] * sc
            return 0
        lax.fori_loop(0, _CH, _edge_scale, 0, unroll=4)

        for g in range(_CH // 16):
            dstc[pl.ds(g * 16, 16)] = dst_v[pl.ds(cb + g * 16, 16)]
        pltpu.async_copy(outc, agg_sh.at[dstc], ssc, add=True)

    _fire(0, 0)
    _fire(1, 1)

    def _ring_body(g, _):
        c0 = 2 * g
        _wait(0, c0)
        _process(0, c0)

        @pl.when(c0 + 2 < _NCH)
        def _():
            _fire(0, c0 + 2)

        c1 = 2 * g + 1
        _wait(1, c1)
        _process(1, c1)

        @pl.when(c1 + 2 < _NCH)
        def _():
            _fire(1, c1 + 2)
        return 0

    lax.fori_loop(0, _NCH // 2, _ring_body, 0)

    pltpu.make_async_copy(out0, agg_sh.at[dst0], ssc0).wait()
    pltpu.make_async_copy(out1, agg_sh.at[dst1], ssc1).wait()

    plsc.subcore_barrier()
    rb = sid * _RPS
    pltpu.sync_copy(agg_sh.at[pl.ds(rb, _RPS)],
                    out_hbm.at[cid, pl.ds(rb, _RPS)])

    @pl.when(sid == _NS - 1)
    def _flush_tail():
        pltpu.sync_copy(agg_sh.at[pl.ds(_RPS * _NS, _N_TAIL)],
                        out_hbm.at[cid, pl.ds(_RPS * _NS, _N_TAIL)])


@functools.partial(jax.jit, static_argnames=())
def _edge_phase(hW_flat, comb_flat, hA2, gidx_p, dst_p, norm_p, bw, bb16):
    mesh = plsc.VectorSubcoreMesh(core_axis_name="c", subcore_axis_name="s")
    f32 = jnp.float32
    i32 = jnp.int32
    kern = functools.partial(
        pl.kernel,
        mesh=mesh,
        compiler_params=pltpu.CompilerParams(needs_layout_passes=False),
        out_type=jax.ShapeDtypeStruct((_NC, _N, _EMB), f32),
        scratch_types=[
            pltpu.VMEM((_EPT,), i32),
            pltpu.VMEM((_EPT,), i32),
            pltpu.VMEM((_CH, _EMB), f32),
            pltpu.VMEM((_CH, _EMB), f32),
            pltpu.VMEM((_CH, _EMB), f32),
            pltpu.VMEM((_CH, _EMB), f32),
            pltpu.VMEM((_CH, _EMB), f32),
            pltpu.VMEM((_CH, _EMB), f32),
            pltpu.VMEM((_CH, _EMB), f32),
            pltpu.VMEM((_CH, _EMB), f32),
            pltpu.VMEM((_CH,), i32),
            pltpu.VMEM((_CH,), i32),
            pltpu.VMEM((_CH,), f32),
            pltpu.VMEM((_CH,), f32),
            pltpu.VMEM((_CH,), f32),
            pltpu.VMEM((16, 16), f32),
            pltpu.VMEM((_EMB,), f32),
            pltpu.VMEM((16,), f32),
            pltpu.VMEM((8, _EMB), f32),
            pltpu.VMEM_SHARED((_N, _EMB), f32),
            pltpu.SemaphoreType.DMA,
            pltpu.SemaphoreType.DMA,
            pltpu.SemaphoreType.DMA,
            pltpu.SemaphoreType.DMA,
            pltpu.SemaphoreType.DMA,
            pltpu.SemaphoreType.DMA,
            pltpu.SemaphoreType.DMA,
            pltpu.SemaphoreType.DMA,
            pltpu.SemaphoreType.DMA,
            pltpu.SemaphoreType.DMA,
        ],
    )(_edge_sc_kernel)
    return kern(hW_flat, comb_flat, hA2, gidx_p, dst_p, norm_p, bw, bb16)


def kernel(x, edge_index, edge_type, norm, basis, w_comp, w_self, bias, A_w, A_b, B_w, B_b, attn_emb, bn_gamma, bn_beta):
    pad = _E_PAD - _E
    src_p = jnp.pad(edge_index[0].astype(jnp.int32), (0, pad))
    dst_p = jnp.pad(edge_index[1].astype(jnp.int32), (0, pad))
    typ_p = jnp.pad(edge_type.astype(jnp.int32), (0, pad))
    gidx_p = typ_p * _N + src_p
    norm_p = jnp.pad(norm[:, 0], (0, pad))

    h = x
    h_in = x
    for l in range(_NUM_LAYERS):
        if l > 0:
            h_in = h
        weight = jnp.einsum('rb,bio->rio', w_comp[l, :_NUM_RELS], basis[l])
        A1 = A_w[l, :_EMB]
        A2 = A_w[l, _EMB:2 * _EMB]
        A3 = A_w[l, 2 * _EMB:2 * _EMB + _ATTN]
        A4 = A_w[l, 2 * _EMB + _ATTN:]
        w_all = jnp.concatenate(
            [weight, A1[None], A2[None], w_self[l][None]], axis=0)
        y = _stacked_matmul(h, w_all)
        hW_flat = y[:_NUM_RELS].reshape(_NUM_RELS * _N, _EMB)
        hA1 = y[_NUM_RELS]
        hA2 = y[_NUM_RELS + 1]
        hs = y[_NUM_RELS + 2]
        relb = attn_emb @ (A3 + A4) + A_b[l]
        comb = _build_comb(hA1, relb).reshape(_NUM_RELS * _N, _EMB)
        bb16 = jnp.full((16,), B_b[l, 0], jnp.float32)
        agg2 = _edge_phase(hW_flat, comb, hA2, gidx_p, dst_p, norm_p,
                           B_w[l, :, 0], bb16)
        h = _post_layer(hs, agg2, bias[l])

    stats = _bn_stats(h)[0]
    mean = stats[0] / _N
    var = stats[1] / _N - mean * mean
    inv = bn_gamma / jnp.sqrt(var + 1e-5)
    scale = inv
    shift = bn_beta - mean * inv
    return _bn_apply(h, h_in, scale, shift)


# speedup vs baseline: 1.0530x; 1.0530x over previous
import functools
import jax
import jax.numpy as jnp
from jax import lax
from jax.experimental import pallas as pl
from jax.experimental.pallas import tpu as pltpu, tpu_sc as plsc

_N = 10000
_E = 160000
_EMB = 128
_ATTN = 32
_NUM_RELS = 8
_NUM_LAYERS = 3

_ROWS_BLK = 1000
_NB = _N // _ROWS_BLK

_NC = 2
_NS = 16
_NW = _NC * _NS
_CH = 32
_NCH = 158
_EPT = _CH * _NCH
_E_PAD = _EPT * _NW
_RPS = 624
_N_TAIL = _N - _RPS * _NS


def _mm_kernel(h_ref, w_ref, o_ref):
    o_ref[0] = jnp.dot(h_ref[...], w_ref[0], preferred_element_type=jnp.float32)


def _stacked_matmul(h, w_all):
    c = w_all.shape[0]
    return pl.pallas_call(
        _mm_kernel,
        grid=(c, _NB),
        in_specs=[
            pl.BlockSpec((_ROWS_BLK, _EMB), lambda i, j: (j, 0)),
            pl.BlockSpec((1, _EMB, _EMB), lambda i, j: (i, 0, 0)),
        ],
        out_specs=pl.BlockSpec((1, _ROWS_BLK, _EMB), lambda i, j: (i, j, 0)),
        out_shape=jax.ShapeDtypeStruct((c, _N, _EMB), jnp.float32),
    )(h, w_all)


def _comb_kernel(a1_ref, relb_ref, o_ref):
    o_ref[0] = a1_ref[...] + relb_ref[0, 0]


def _build_comb(hA1, relb):
    return pl.pallas_call(
        _comb_kernel,
        grid=(_NUM_RELS, _NB),
        in_specs=[
            pl.BlockSpec((_ROWS_BLK, _EMB), lambda i, j: (j, 0)),
            pl.BlockSpec((1, 1, _EMB), lambda i, j: (i, 0, 0)),
        ],
        out_specs=pl.BlockSpec((1, _ROWS_BLK, _EMB), lambda i, j: (i, j, 0)),
        out_shape=jax.ShapeDtypeStruct((_NUM_RELS, _N, _EMB), jnp.float32),
    )(hA1, relb[:, None, :])


def _post_kernel(hs_ref, agg_ref, bias_ref, o_ref):
    o_ref[...] = jnp.maximum(
        hs_ref[...] + agg_ref[0, 0] + agg_ref[0, 1] + bias_ref[...], 0.0)


def _post_layer(hs, agg2, bias_l):
    return pl.pallas_call(
        _post_kernel,
        grid=(_NB,),
        in_specs=[
            pl.BlockSpec((_ROWS_BLK, _EMB), lambda j: (j, 0)),
            pl.BlockSpec((1, 2, _ROWS_BLK, _EMB), lambda j: (0, 0, j, 0)),
            pl.BlockSpec((1, _EMB), lambda j: (0, 0)),
        ],
        out_specs=pl.BlockSpec((_ROWS_BLK, _EMB), lambda j: (j, 0)),
        out_shape=jax.ShapeDtypeStruct((_N, _EMB), jnp.float32),
    )(hs, agg2[None], bias_l[None])


def _stats_kernel(h_ref, o_ref):
    @pl.when(pl.program_id(0) == 0)
    def _():
        o_ref[...] = jnp.zeros_like(o_ref)
    blk = h_ref[...]
    o_ref[0, 0] += jnp.sum(blk, axis=0)
    o_ref[0, 1] += jnp.sum(blk * blk, axis=0)


def _bn_stats(h):
    return pl.pallas_call(
        _stats_kernel,
        grid=(_NB,),
        in_specs=[pl.BlockSpec((_ROWS_BLK, _EMB), lambda j: (j, 0))],
        out_specs=pl.BlockSpec((1, 2, _EMB), lambda j: (0, 0, 0)),
        out_shape=jax.ShapeDtypeStruct((1, 2, _EMB), jnp.float32),
    )(h)


def _bn_apply_kernel(h_ref, hin_ref, scale_ref, shift_ref, o_ref):
    o_ref[...] = hin_ref[...] + h_ref[...] * scale_ref[0] + shift_ref[0]


def _bn_apply(h, h_in, scale, shift):
    return pl.pallas_call(
        _bn_apply_kernel,
        grid=(_NB,),
        in_specs=[
            pl.BlockSpec((_ROWS_BLK, _EMB), lambda j: (j, 0)),
            pl.BlockSpec((_ROWS_BLK, _EMB), lambda j: (j, 0)),
            pl.BlockSpec((1, _EMB), lambda j: (0, 0)),
            pl.BlockSpec((1, _EMB), lambda j: (0, 0)),
        ],
        out_specs=pl.BlockSpec((_ROWS_BLK, _EMB), lambda j: (j, 0)),
        out_shape=jax.ShapeDtypeStruct((_N, _EMB), jnp.float32),
    )(h, h_in, scale[None], shift[None])


_G16 = _EMB // 16


def _edge_sc_kernel(hW_hbm, comb_hbm, hA2_hbm, gidx_hbm, dst_hbm,
                    norm_hbm, bw_hbm, bb_hbm, out_hbm,
                    gidx_v, dst_v,
                    msg0, msg1, a10, a11, a20, a21, out0, out1,
                    dst0, dst1, nrm0, nrm1, tbuf, vbuf,
                    bw_v, bb_v, zbuf, agg_sh,
                    s0a, s0b, s0c, s0d, s1a, s1b, s1c, s1d, ssc0, ssc1):
    cid = lax.axis_index("c")
    sid = lax.axis_index("s")
    wid = sid * _NC + cid
    ebase = wid * _EPT

    pltpu.sync_copy(gidx_hbm.at[pl.ds(ebase, _EPT)], gidx_v)
    pltpu.sync_copy(dst_hbm.at[pl.ds(ebase, _EPT)], dst_v)
    pltpu.sync_copy(bw_hbm, bw_v)
    pltpu.sync_copy(bb_hbm, bb_v)

    def _zb_body(k, _):
        for j in range(_G16):
            zbuf[k, pl.ds(j * 16, 16)] = jnp.zeros((16,), jnp.float32)
        return 0
    lax.fori_loop(0, 8, _zb_body, 0)

    def _zc_body(k, _):
        pltpu.sync_copy(zbuf, agg_sh.at[pl.ds(sid * _RPS + k * 8, 8)])
        return 0
    lax.fori_loop(0, _RPS // 8, _zc_body, 0)

    @pl.when(sid == _NS - 1)
    def _zero_tail():
        pltpu.sync_copy(zbuf, agg_sh.at[pl.ds(_RPS * _NS, 8)])
        pltpu.sync_copy(zbuf, agg_sh.at[pl.ds(_RPS * _NS + 8, 8)])
    plsc.subcore_barrier()

    bwv = [bw_v[pl.ds(j * 16, 16)] for j in range(_G16)]
    bbv = bb_v[...]
    eidx = lax.iota(jnp.int32, 16)
    msgb = (msg0, msg1)
    a1b = (a10, a11)
    a2b = (a20, a21)
    outb = (out0, out1)
    dstb = (dst0, dst1)
    nrmb = (nrm0, nrm1)
    sscb = (ssc0, ssc1)
    sems = ((s0a, s0b, s0c, s0d), (s1a, s1b, s1c, s1d))

    def _fire(b, ci):
        cb = ci * _CH
        idx_m = gidx_v.at[pl.ds(cb, _CH)]
        idx_d = dst_v.at[pl.ds(cb, _CH)]
        pltpu.async_copy(hW_hbm.at[idx_m], msgb[b], sems[b][0])
        pltpu.async_copy(comb_hbm.at[idx_m], a1b[b], sems[b][1])
        pltpu.async_copy(hA2_hbm.at[idx_d], a2b[b], sems[b][2])
        pltpu.async_copy(norm_hbm.at[pl.ds(ebase + cb, _CH)], nrmb[b],
                         sems[b][3])

    def _wait(b, ci):
        cb = ci * _CH
        idx_m = gidx_v.at[pl.ds(cb, _CH)]
        idx_d = dst_v.at[pl.ds(cb, _CH)]
        pltpu.make_async_copy(hW_hbm.at[idx_m], msgb[b], sems[b][0]).wait()
        pltpu.make_async_copy(comb_hbm.at[idx_m], a1b[b], sems[b][1]).wait()
        pltpu.make_async_copy(hA2_hbm.at[idx_d], a2b[b], sems[b][2]).wait()
        pltpu.make_async_copy(norm_hbm.at[pl.ds(ebase + cb, _CH)], nrmb[b],
                              sems[b][3]).wait()

    def _process(b, ci):
        msgc, a1c, a2c = msgb[b], a1b[b], a2b[b]
        outc, dstc, ssc, nrmc = outb[b], dstb[b], sscb[b], nrmb[b]
        cb = ci * _CH
        for g in range(_CH // 16):
            def _edge_dot(ee, _, g=g):
                e = g * 16 + ee
                vacc = jnp.zeros((16,), jnp.float32)
                for j in range(_G16):
                    sl = pl.ds(j * 16, 16)
                    pre = jnp.maximum(a1c[e, sl] + a2c[e, sl], 0.0)
                    vacc = vacc + pre * bwv[j]
                vbuf[ee, :] = vacc
                return 0
            lax.fori_loop(0, 16, _edge_dot, 0)
            tv = jnp.zeros((16,), jnp.float32)
            for j in range(16):
                tv = tv + plsc.load_gather(
                    vbuf, [eidx, jnp.full((16,), j, jnp.int32)])
            av = 1.0 / (1.0 + jnp.exp(-(tv + bbv)))
            tbuf[pl.ds(g * 16, 16)] = av * nrmc[pl.ds(g * 16, 16)]

        @pl.when(ci >= 2)
        def _():
            pltpu.make_async_copy(outc, agg_sh.at[dstc], ssc).wait()

        def _edge_scale(e, _):
            sc = plsc.load_gather(tbuf, [jnp.full((16,), e, jnp.int32)])
            for j in range(_G16):
                sl = pl.ds(j * 16, 16)
                outc[e, sl] = msgc[e, sl] * sc
            return 0
        lax.fori_loop(0, _CH, _edge_scale, 0)

        for g in range(_CH // 16):
            dstc[pl.ds(g * 16, 16)] = dst_v[pl.ds(cb + g * 16, 16)]
        pltpu.async_copy(outc, agg_sh.at[dstc], ssc, add=True)

    _fire(0, 0)
    _fire(1, 1)

    def _ring_body(g, _):
        c0 = 2 * g
        _wait(0, c0)
        _process(0, c0)

        @pl.when(c0 + 2 < _NCH)
        def _():
            _fire(0, c0 + 2)

        c1 = 2 * g + 1
        _wait(1, c1)
        _process(1, c1)

        @pl.when(c1 + 2 < _NCH)
        def _():
            _fire(1, c1 + 2)
        return 0

    lax.fori_loop(0, _NCH // 2, _ring_body, 0)

    pltpu.make_async_copy(out0, agg_sh.at[dst0], ssc0).wait()
    pltpu.make_async_copy(out1, agg_sh.at[dst1], ssc1).wait()

    plsc.subcore_barrier()
    rb = sid * _RPS
    pltpu.sync_copy(agg_sh.at[pl.ds(rb, _RPS)],
                    out_hbm.at[cid, pl.ds(rb, _RPS)])

    @pl.when(sid == _NS - 1)
    def _flush_tail():
        pltpu.sync_copy(agg_sh.at[pl.ds(_RPS * _NS, _N_TAIL)],
                        out_hbm.at[cid, pl.ds(_RPS * _NS, _N_TAIL)])


@functools.partial(jax.jit, static_argnames=())
def _edge_phase(hW_flat, comb_flat, hA2, gidx_p, dst_p, norm_p, bw, bb16):
    mesh = plsc.VectorSubcoreMesh(core_axis_name="c", subcore_axis_name="s")
    f32 = jnp.float32
    i32 = jnp.int32
    kern = functools.partial(
        pl.kernel,
        mesh=mesh,
        compiler_params=pltpu.CompilerParams(needs_layout_passes=False),
        out_type=jax.ShapeDtypeStruct((_NC, _N, _EMB), f32),
        scratch_types=[
            pltpu.VMEM((_EPT,), i32),
            pltpu.VMEM((_EPT,), i32),
            pltpu.VMEM((_CH, _EMB), f32),
            pltpu.VMEM((_CH, _EMB), f32),
            pltpu.VMEM((_CH, _EMB), f32),
            pltpu.VMEM((_CH, _EMB), f32),
            pltpu.VMEM((_CH, _EMB), f32),
            pltpu.VMEM((_CH, _EMB), f32),
            pltpu.VMEM((_CH, _EMB), f32),
            pltpu.VMEM((_CH, _EMB), f32),
            pltpu.VMEM((_CH,), i32),
            pltpu.VMEM((_CH,), i32),
            pltpu.VMEM((_CH,), f32),
            pltpu.VMEM((_CH,), f32),
            pltpu.VMEM((_CH,), f32),
            pltpu.VMEM((16, 16), f32),
            pltpu.VMEM((_EMB,), f32),
            pltpu.VMEM((16,), f32),
            pltpu.VMEM((8, _EMB), f32),
            pltpu.VMEM_SHARED((_N, _EMB), f32),
            pltpu.SemaphoreType.DMA,
            pltpu.SemaphoreType.DMA,
            pltpu.SemaphoreType.DMA,
            pltpu.SemaphoreType.DMA,
            pltpu.SemaphoreType.DMA,
            pltpu.SemaphoreType.DMA,
            pltpu.SemaphoreType.DMA,
            pltpu.SemaphoreType.DMA,
            pltpu.SemaphoreType.DMA,
            pltpu.SemaphoreType.DMA,
        ],
    )(_edge_sc_kernel)
    return kern(hW_flat, comb_flat, hA2, gidx_p, dst_p, norm_p, bw, bb16)


def kernel(x, edge_index, edge_type, norm, basis, w_comp, w_self, bias, A_w, A_b, B_w, B_b, attn_emb, bn_gamma, bn_beta):
    pad = _E_PAD - _E
    src_p = jnp.pad(edge_index[0].astype(jnp.int32), (0, pad))
    dst_p = jnp.pad(edge_index[1].astype(jnp.int32), (0, pad))
    typ_p = jnp.pad(edge_type.astype(jnp.int32), (0, pad))
    gidx_p = typ_p * _N + src_p
    norm_p = jnp.pad(norm[:, 0], (0, pad))

    h = x
    h_in = x
    for l in range(_NUM_LAYERS):
        if l > 0:
            h_in = h
        weight = jnp.einsum('rb,bio->rio', w_comp[l, :_NUM_RELS], basis[l])
        A1 = A_w[l, :_EMB]
        A2 = A_w[l, _EMB:2 * _EMB]
        A3 = A_w[l, 2 * _EMB:2 * _EMB + _ATTN]
        A4 = A_w[l, 2 * _EMB + _ATTN:]
        w_all = jnp.concatenate(
            [weight, A1[None], A2[None], w_self[l][None]], axis=0)
        y = _stacked_matmul(h, w_all)
        hW_flat = y[:_NUM_RELS].reshape(_NUM_RELS * _N, _EMB)
        hA1 = y[_NUM_RELS]
        hA2 = y[_NUM_RELS + 1]
        hs = y[_NUM_RELS + 2]
        relb = attn_emb @ (A3 + A4) + A_b[l]
        comb = _build_comb(hA1, relb).reshape(_NUM_RELS * _N, _EMB)
        bb16 = jnp.full((16,), B_b[l, 0], jnp.float32)
        agg2 = _edge_phase(hW_flat, comb, hA2, gidx_p, dst_p, norm_p,
                           B_w[l, :, 0], bb16)
        h = _post_layer(hs, agg2, bias[l])

    stats = _bn_stats(h)[0]
    mean = stats[0] / _N
    var = stats[1] / _N - mean * mean
    inv = bn_gamma / jnp.sqrt(var + 1e-5)
    scale = inv
    shift = bn_beta - mean * inv
    return _bn_apply(h, h_in, scale, shift)


# speedup vs baseline: 1.2784x; 1.2141x over previous
import functools
import jax
import jax.numpy as jnp
from jax import lax
from jax.experimental import pallas as pl
from jax.experimental.pallas import tpu as pltpu, tpu_sc as plsc

_N = 10000
_E = 160000
_EMB = 128
_ATTN = 32
_NUM_RELS = 8
_NUM_LAYERS = 3

_ROWS_BLK = 1000
_NB = _N // _ROWS_BLK

_NC = 2
_NS = 16
_NW = _NC * _NS
_CH = 32
_NCH = 158
_EPT = _CH * _NCH
_E_PAD = _EPT * _NW
_RPS = 624
_N_TAIL = _N - _RPS * _NS


def _mm_kernel(h_ref, w_ref, o_ref):
    o_ref[0] = jnp.dot(h_ref[...], w_ref[0], preferred_element_type=jnp.float32)


def _stacked_matmul(h, w_all):
    c = w_all.shape[0]
    return pl.pallas_call(
        _mm_kernel,
        grid=(c, _NB),
        in_specs=[
            pl.BlockSpec((_ROWS_BLK, _EMB), lambda i, j: (j, 0)),
            pl.BlockSpec((1, _EMB, _EMB), lambda i, j: (i, 0, 0)),
        ],
        out_specs=pl.BlockSpec((1, _ROWS_BLK, _EMB), lambda i, j: (i, j, 0)),
        out_shape=jax.ShapeDtypeStruct((c, _N, _EMB), jnp.float32),
    )(h, w_all)


def _comb_kernel(a1_ref, relb_ref, o_ref):
    o_ref[0] = a1_ref[...] + relb_ref[0, 0]


def _build_comb(hA1, relb):
    return pl.pallas_call(
        _comb_kernel,
        grid=(_NUM_RELS, _NB),
        in_specs=[
            pl.BlockSpec((_ROWS_BLK, _EMB), lambda i, j: (j, 0)),
            pl.BlockSpec((1, 1, _EMB), lambda i, j: (i, 0, 0)),
        ],
        out_specs=pl.BlockSpec((1, _ROWS_BLK, _EMB), lambda i, j: (i, j, 0)),
        out_shape=jax.ShapeDtypeStruct((_NUM_RELS, _N, _EMB), jnp.float32),
    )(hA1, relb[:, None, :])


def _post_kernel(hs_ref, agg_ref, bias_ref, o_ref):
    o_ref[...] = jnp.maximum(
        hs_ref[...] + agg_ref[0, 0] + agg_ref[0, 1] + bias_ref[...], 0.0)


def _post_layer(hs, agg2, bias_l):
    return pl.pallas_call(
        _post_kernel,
        grid=(_NB,),
        in_specs=[
            pl.BlockSpec((_ROWS_BLK, _EMB), lambda j: (j, 0)),
            pl.BlockSpec((1, 2, _ROWS_BLK, _EMB), lambda j: (0, 0, j, 0)),
            pl.BlockSpec((1, _EMB), lambda j: (0, 0)),
        ],
        out_specs=pl.BlockSpec((_ROWS_BLK, _EMB), lambda j: (j, 0)),
        out_shape=jax.ShapeDtypeStruct((_N, _EMB), jnp.float32),
    )(hs, agg2[None], bias_l[None])


def _stats_kernel(h_ref, o_ref):
    @pl.when(pl.program_id(0) == 0)
    def _():
        o_ref[...] = jnp.zeros_like(o_ref)
    blk = h_ref[...]
    o_ref[0, 0] += jnp.sum(blk, axis=0)
    o_ref[0, 1] += jnp.sum(blk * blk, axis=0)


def _bn_stats(h):
    return pl.pallas_call(
        _stats_kernel,
        grid=(_NB,),
        in_specs=[pl.BlockSpec((_ROWS_BLK, _EMB), lambda j: (j, 0))],
        out_specs=pl.BlockSpec((1, 2, _EMB), lambda j: (0, 0, 0)),
        out_shape=jax.ShapeDtypeStruct((1, 2, _EMB), jnp.float32),
    )(h)


def _bn_apply_kernel(h_ref, hin_ref, scale_ref, shift_ref, o_ref):
    o_ref[...] = hin_ref[...] + h_ref[...] * scale_ref[0] + shift_ref[0]


def _bn_apply(h, h_in, scale, shift):
    return pl.pallas_call(
        _bn_apply_kernel,
        grid=(_NB,),
        in_specs=[
            pl.BlockSpec((_ROWS_BLK, _EMB), lambda j: (j, 0)),
            pl.BlockSpec((_ROWS_BLK, _EMB), lambda j: (j, 0)),
            pl.BlockSpec((1, _EMB), lambda j: (0, 0)),
            pl.BlockSpec((1, _EMB), lambda j: (0, 0)),
        ],
        out_specs=pl.BlockSpec((_ROWS_BLK, _EMB), lambda j: (j, 0)),
        out_shape=jax.ShapeDtypeStruct((_N, _EMB), jnp.float32),
    )(h, h_in, scale[None], shift[None])


_G16 = _EMB // 16


def _edge_sc_kernel(hW_hbm, comb_hbm, hA2_hbm, gidx_hbm, dst_hbm,
                    norm_hbm, bw_hbm, bb_hbm, out_hbm,
                    gidx_v, dst_v, norm_v,
                    msg0, msg1, a10, a11, a20, a21, dstc, tbuf, vbuf,
                    bw_v, bb_v, zbuf, agg_sh,
                    s0a, s0b, s0c, s1a, s1b, s1c):
    cid = lax.axis_index("c")
    sid = lax.axis_index("s")
    wid = sid * _NC + cid
    ebase = wid * _EPT

    pltpu.sync_copy(gidx_hbm.at[pl.ds(ebase, _EPT)], gidx_v)
    pltpu.sync_copy(dst_hbm.at[pl.ds(ebase, _EPT)], dst_v)
    pltpu.sync_copy(norm_hbm.at[pl.ds(ebase, _EPT)], norm_v)
    pltpu.sync_copy(bw_hbm, bw_v)
    pltpu.sync_copy(bb_hbm, bb_v)

    def _zb_body(k, _):
        for j in range(_G16):
            zbuf[k, pl.ds(j * 16, 16)] = jnp.zeros((16,), jnp.float32)
        return 0
    lax.fori_loop(0, 16, _zb_body, 0)

    def _zc_body(k, _):
        pltpu.sync_copy(zbuf, agg_sh.at[pl.ds(sid * _RPS + k * 16, 16)])
        return 0
    lax.fori_loop(0, _RPS // 16, _zc_body, 0)

    @pl.when(sid == _NS - 1)
    def _zero_tail():
        pltpu.sync_copy(zbuf, agg_sh.at[pl.ds(_RPS * _NS, _N_TAIL)])
    plsc.subcore_barrier()

    bwv = [bw_v[pl.ds(j * 16, 16)] for j in range(_G16)]
    bbv = bb_v[...]
    eidx = lax.iota(jnp.int32, 16)
    msgb = (msg0, msg1)
    a1b = (a10, a11)
    a2b = (a20, a21)
    sems = ((s0a, s0b, s0c), (s1a, s1b, s1c))

    def _fire(b, ci):
        cb = ci * _CH
        idx_m = gidx_v.at[pl.ds(cb, _CH)]
        idx_d = dst_v.at[pl.ds(cb, _CH)]
        return (pltpu.async_copy(hW_hbm.at[idx_m], msgb[b], sems[b][0]),
                pltpu.async_copy(comb_hbm.at[idx_m], a1b[b], sems[b][1]),
                pltpu.async_copy(hA2_hbm.at[idx_d], a2b[b], sems[b][2]))

    def _wait(b, ci):
        cb = ci * _CH
        idx_m = gidx_v.at[pl.ds(cb, _CH)]
        idx_d = dst_v.at[pl.ds(cb, _CH)]
        pltpu.make_async_copy(hW_hbm.at[idx_m], msgb[b], sems[b][0]).wait()
        pltpu.make_async_copy(comb_hbm.at[idx_m], a1b[b], sems[b][1]).wait()
        pltpu.make_async_copy(hA2_hbm.at[idx_d], a2b[b], sems[b][2]).wait()

    def _process(b, ci):
        msgc, a1c, a2c = msgb[b], a1b[b], a2b[b]
        cb = ci * _CH
        for g in range(_CH // 16):
            def _edge_dot(ee, _, g=g):
                e = g * 16 + ee
                vacc = jnp.zeros((16,), jnp.float32)
                for j in range(_G16):
                    sl = pl.ds(j * 16, 16)
                    pre = jnp.maximum(a1c[e, sl] + a2c[e, sl], 0.0)
                    vacc = vacc + pre * bwv[j]
                vbuf[ee, :] = vacc
                return 0
            lax.fori_loop(0, 16, _edge_dot, 0)
            tv = jnp.zeros((16,), jnp.float32)
            for j in range(16):
                tv = tv + plsc.load_gather(
                    vbuf, [eidx, jnp.full((16,), j, jnp.int32)])
            av = 1.0 / (1.0 + jnp.exp(-(tv + bbv)))
            tbuf[pl.ds(g * 16, 16)] = av * norm_v[pl.ds(cb + g * 16, 16)]

        def _edge_scale(e, _):
            sc = plsc.load_gather(tbuf, [jnp.full((16,), e, jnp.int32)])
            for j in range(_G16):
                sl = pl.ds(j * 16, 16)
                msgc[e, sl] = msgc[e, sl] * sc
            return 0
        lax.fori_loop(0, _CH, _edge_scale, 0)

        for g in range(_CH // 16):
            dstc[pl.ds(g * 16, 16)] = dst_v[pl.ds(cb + g * 16, 16)]
        pltpu.sync_copy(msgc, agg_sh.at[dstc], add=True)

    _fire(0, 0)
    _fire(1, 1)

    def _ring_body(g, _):
        c0 = 2 * g
        _wait(0, c0)
        _process(0, c0)

        @pl.when(c0 + 2 < _NCH)
        def _():
            _fire(0, c0 + 2)

        c1 = 2 * g + 1
        _wait(1, c1)
        _process(1, c1)

        @pl.when(c1 + 2 < _NCH)
        def _():
            _fire(1, c1 + 2)
        return 0

    lax.fori_loop(0, _NCH // 2, _ring_body, 0)

    plsc.subcore_barrier()
    rb = sid * _RPS
    pltpu.sync_copy(agg_sh.at[pl.ds(rb, _RPS)],
                    out_hbm.at[cid, pl.ds(rb, _RPS)])

    @pl.when(sid == _NS - 1)
    def _flush_tail():
        pltpu.sync_copy(agg_sh.at[pl.ds(_RPS * _NS, _N_TAIL)],
                        out_hbm.at[cid, pl.ds(_RPS * _NS, _N_TAIL)])


@functools.partial(jax.jit, static_argnames=())
def _edge_phase(hW_flat, comb_flat, hA2, gidx_p, dst_p, norm_p, bw, bb16):
    mesh = plsc.VectorSubcoreMesh(core_axis_name="c", subcore_axis_name="s")
    f32 = jnp.float32
    i32 = jnp.int32
    kern = functools.partial(
        pl.kernel,
        mesh=mesh,
        compiler_params=pltpu.CompilerParams(needs_layout_passes=False),
        out_type=jax.ShapeDtypeStruct((_NC, _N, _EMB), f32),
        scratch_types=[
            pltpu.VMEM((_EPT,), i32),
            pltpu.VMEM((_EPT,), i32),
            pltpu.VMEM((_EPT,), f32),
            pltpu.VMEM((_CH, _EMB), f32),
            pltpu.VMEM((_CH, _EMB), f32),
            pltpu.VMEM((_CH, _EMB), f32),
            pltpu.VMEM((_CH, _EMB), f32),
            pltpu.VMEM((_CH, _EMB), f32),
            pltpu.VMEM((_CH, _EMB), f32),
            pltpu.VMEM((_CH,), i32),
            pltpu.VMEM((_CH,), f32),
            pltpu.VMEM((16, 16), f32),
            pltpu.VMEM((_EMB,), f32),
            pltpu.VMEM((16,), f32),
            pltpu.VMEM((16, _EMB), f32),
            pltpu.VMEM_SHARED((_N, _EMB), f32),
            pltpu.SemaphoreType.DMA,
            pltpu.SemaphoreType.DMA,
            pltpu.SemaphoreType.DMA,
            pltpu.SemaphoreType.DMA,
            pltpu.SemaphoreType.DMA,
            pltpu.SemaphoreType.DMA,
        ],
    )(_edge_sc_kernel)
    return kern(hW_flat, comb_flat, hA2, gidx_p, dst_p, norm_p, bw, bb16)


def kernel(x, edge_index, edge_type, norm, basis, w_comp, w_self, bias, A_w, A_b, B_w, B_b, attn_emb, bn_gamma, bn_beta):
    pad = _E_PAD - _E
    src_p = jnp.pad(edge_index[0].astype(jnp.int32), (0, pad))
    dst_p = jnp.pad(edge_index[1].astype(jnp.int32), (0, pad))
    typ_p = jnp.pad(edge_type.astype(jnp.int32), (0, pad))
    gidx_p = typ_p * _N + src_p
    norm_p = jnp.pad(norm[:, 0], (0, pad))

    h = x
    h_in = x
    for l in range(_NUM_LAYERS):
        if l > 0:
            h_in = h
        weight = jnp.einsum('rb,bio->rio', w_comp[l, :_NUM_RELS], basis[l])
        A1 = A_w[l, :_EMB]
        A2 = A_w[l, _EMB:2 * _EMB]
        A3 = A_w[l, 2 * _EMB:2 * _EMB + _ATTN]
        A4 = A_w[l, 2 * _EMB + _ATTN:]
        w_all = jnp.concatenate(
            [weight, A1[None], A2[None], w_self[l][None]], axis=0)
        y = _stacked_matmul(h, w_all)
        hW_flat = y[:_NUM_RELS].reshape(_NUM_RELS * _N, _EMB)
        hA1 = y[_NUM_RELS]
        hA2 = y[_NUM_RELS + 1]
        hs = y[_NUM_RELS + 2]
        relb = attn_emb @ (A3 + A4) + A_b[l]
        comb = _build_comb(hA1, relb).reshape(_NUM_RELS * _N, _EMB)
        bb16 = jnp.full((16,), B_b[l, 0], jnp.float32)
        agg2 = _edge_phase(hW_flat, comb, hA2, gidx_p, dst_p, norm_p,
                           B_w[l, :, 0], bb16)
        h = _post_layer(hs, agg2, bias[l])

    stats = _bn_stats(h)[0]
    mean = stats[0] / _N
    var = stats[1] / _N - mean * mean
    inv = bn_gamma / jnp.sqrt(var + 1e-5)
    scale = inv
    shift = bn_beta - mean * inv
    return _bn_apply(h, h_in, scale, shift)
